# Initial kernel scaffold; baseline (speedup 1.0000x reference)
import jax
import jax.numpy as jnp
from jax.experimental import pallas as pl


def kernel(x):
    raise NotImplementedError("write your pallas kernel here")


# speedup vs baseline: 4.4432x; 4.4432x over previous
import jax
import jax.numpy as jnp
from jax.experimental import pallas as pl
from jax.experimental.pallas import tpu as pltpu


def _chunk_sizes():
    sizes = []
    total = 0
    i = 0
    while total < 6400:
        s = 2 + (i % 8)
        sizes.append(s)
        total += s
        i += 1
    return sizes


_SIZES = _chunk_sizes()
_OFFSETS = [0]
for _s in _SIZES:
    _OFFSETS.append(_OFFSETS[-1] + _s)


def _split_body(x_ref, *out_refs):
    for i, o in enumerate(out_refs):
        off = _OFFSETS[i]
        s = _SIZES[i]
        o[...] = x_ref[off:off + s, :]


def kernel(x):
    out_shape = [jax.ShapeDtypeStruct((s, x.shape[1]), x.dtype) for s in _SIZES]
    outs = pl.pallas_call(_split_body, out_shape=out_shape)(x)
    return tuple(outs)
